# baseline (speedup 1.0000x reference)
import functools

import jax
import jax.numpy as jnp
from jax import lax
from jax.experimental import pallas as pl
from jax.experimental.pallas import tpu as pltpu
from jax.experimental.pallas import tpu_sc as plsc

NC, NS = 2, 16
NW = NC * NS
GC = 128
NCH = 82
CE = NCH * GC
EPAD = NW * CE
NPAD = 10240
STR = NPAD // NS
EPS = 1e-16

_mesh = functools.partial(plsc.VectorSubcoreMesh,
                          core_axis_name="c", subcore_axis_name="s")
_sc_params = pltpu.CompilerParams(needs_layout_passes=False,
                                  use_tc_tiling_on_sc=False)


def _ids():
    cid = lax.axis_index("c")
    sid = lax.axis_index("s")
    return cid, sid, sid * NC + cid


def _zero16():
    return jnp.zeros((16,), jnp.float32)


def _stripe_merge(part_v, sh, sid, stripe_v):
    pltpu.sync_copy(part_v, sh.at[sid])
    plsc.subcore_barrier()
    pltpu.sync_copy(sh.at[:, pl.ds(sid * STR, STR)], stripe_v)

    def _sum(i, c):
        sl = pl.ds(i * 16, 16)
        acc = stripe_v[0, sl]
        for t in range(1, NS):
            acc = acc + stripe_v[t, sl]
        part_v[sl] = acc
        return c
    lax.fori_loop(0, STR // 16, _sum, 0)


def _phase_a_body(src_h, dst_h, as_h, ad_h, ex_h, den_h,
                  src_v, dst_v, as_v, ad_v, den_v, ex_v, stripe_v, den_sh):
    cid, sid, wid = _ids()
    pltpu.sync_copy(src_h.at[wid], src_v)
    pltpu.sync_copy(dst_h.at[wid], dst_v)
    pltpu.sync_copy(as_h, as_v)
    pltpu.sync_copy(ad_h, ad_v)

    def _z(i, c):
        den_v[pl.ds(i * 16, 16)] = _zero16()
        return c
    lax.fori_loop(0, NPAD // 16, _z, 0)

    def _step(i, c):
        sl = pl.ds(i * 16, 16)
        s16 = src_v[sl]
        d16 = dst_v[sl]
        a = plsc.load_gather(as_v, [s16])
        b = plsc.load_gather(ad_v, [d16])
        t = a + b
        ex = jnp.exp(jnp.where(t >= 0, t, 0.2 * t))
        ex_v[sl] = ex
        plsc.addupdate_scatter(den_v, [d16], ex)
        return c
    lax.fori_loop(0, CE // 16, _step, 0)

    pltpu.sync_copy(ex_v, ex_h.at[wid])
    _stripe_merge(den_v, den_sh, sid, stripe_v)
    pltpu.sync_copy(den_v.at[pl.ds(0, STR)],
                    den_h.at[cid, pl.ds(sid * STR, STR)])


_phase_a = pl.kernel(
    _phase_a_body,
    out_type=(jax.ShapeDtypeStruct((NW, CE), jnp.float32),
              jax.ShapeDtypeStruct((NC, NPAD), jnp.float32)),
    mesh=_mesh(),
    compiler_params=_sc_params,
    scratch_types=[
        pltpu.VMEM((CE,), jnp.int32),
        pltpu.VMEM((CE,), jnp.int32),
        pltpu.VMEM((NPAD,), jnp.float32),
        pltpu.VMEM((NPAD,), jnp.float32),
        pltpu.VMEM((NPAD,), jnp.float32),
        pltpu.VMEM((CE,), jnp.float32),
        pltpu.VMEM((NS, STR), jnp.float32),
        pltpu.VMEM_SHARED((NS, NPAD), jnp.float32),
    ],
)


def _phase_b_vec_body(hp, stage_h, src3_h, dst_h, ex_h, den_h, h_h, out_h,
                      src3_v, dst_v, ex_v, den_v, den1_v, alpha_v, didx_v,
                      rows0_v, rows1_v, sem0, sem1, *shared):
    if stage_h:
        h_sh, out_sh = shared
    else:
        (out_sh,) = shared
    cid, sid, wid = _ids()
    pltpu.sync_copy(src3_h.at[wid], src3_v)
    pltpu.sync_copy(dst_h.at[wid], dst_v)
    pltpu.sync_copy(ex_h.at[wid], ex_v)
    pltpu.sync_copy(den_h.at[0], den_v)
    pltpu.sync_copy(den_h.at[1], den1_v)

    def _dsum(i, c):
        sl = pl.ds(i * 16, 16)
        den_v[sl] = den_v[sl] + den1_v[sl]
        return c
    lax.fori_loop(0, NPAD // 16, _dsum, 0)

    for e2 in range(GC):
        for c in range(hp // 16):
            rows0_v[e2, pl.ds(c * 16, 16)] = _zero16()
    for i in range(NPAD // GC // NS):
        pltpu.sync_copy(rows0_v, out_sh.at[pl.ds((sid * 5 + i) * GC, GC), :])
    if stage_h:
        @pl.when(sid == 0)
        def _():
            pltpu.sync_copy(h_h, h_sh)
        h_src = h_sh
    else:
        h_src = h_h.at[cid]
    plsc.subcore_barrier()

    rows = (rows0_v, rows1_v)
    sems = (sem0, sem1)
    pltpu.async_copy(h_src.at[src3_v.at[0]], rows0_v, sem0)

    def _pair(g, carry):
        for b in range(2):
            ch = g * 2 + b
            buf = rows[b]

            @pl.when(ch + 1 < NCH)
            def _():
                pltpu.async_copy(h_src.at[src3_v.at[ch + 1]],
                                 rows[1 - b], sems[1 - b])
            pltpu.make_async_copy(h_src.at[src3_v.at[ch]], buf, sems[b]).wait()

            for k in range(GC // 16):
                sl = pl.ds(k * 16, 16)
                esl = pl.ds(ch * GC + k * 16, 16)
                d16 = dst_v[esl]
                g16 = plsc.load_gather(den_v, [d16])
                alpha_v[sl] = ex_v[esl] / (g16 + EPS)
                didx_v[sl] = d16
            for k in range(GC // 16):
                al16 = alpha_v[pl.ds(k * 16, 16)]
                for l in range(16):
                    e2 = k * 16 + l
                    a16 = jnp.full((16,), al16[l], jnp.float32)
                    for c in range(hp // 16):
                        sl2 = pl.ds(c * 16, 16)
                        buf[e2, sl2] = buf[e2, sl2] * a16
            pltpu.sync_copy(buf, out_sh.at[didx_v], add=True)
        return carry
    lax.fori_loop(0, NCH // 2, _pair, 0)

    plsc.subcore_barrier()

    @pl.when(sid == 0)
    def _():
        pltpu.sync_copy(out_sh, out_h.at[cid])


def _make_phase_b_vec(hp, stage_h):
    shared = [pltpu.VMEM_SHARED((NPAD, hp), jnp.float32)]
    if stage_h:
        shared = [pltpu.VMEM_SHARED((NPAD, hp), jnp.float32)] + shared
    return pl.kernel(
        functools.partial(_phase_b_vec_body, hp, stage_h),
        out_type=jax.ShapeDtypeStruct((NC, NPAD, hp), jnp.float32),
        mesh=_mesh(),
        compiler_params=_sc_params,
        scratch_types=[
            pltpu.VMEM((NCH, GC), jnp.int32),
            pltpu.VMEM((CE,), jnp.int32),
            pltpu.VMEM((CE,), jnp.float32),
            pltpu.VMEM((NPAD,), jnp.float32),
            pltpu.VMEM((NPAD,), jnp.float32),
            pltpu.VMEM((GC,), jnp.float32),
            pltpu.VMEM((GC,), jnp.int32),
            pltpu.VMEM((GC, hp), jnp.float32),
            pltpu.VMEM((GC, hp), jnp.float32),
            pltpu.SemaphoreType.DMA,
            pltpu.SemaphoreType.DMA,
        ] + shared,
    )


_phase_b_48 = _make_phase_b_vec(48, True)
_phase_b_64 = _make_phase_b_vec(64, False)


def _phase_b_sc_body(src_h, dst_h, ex_h, den_h, h_h, out_h,
                     src_v, dst_v, ex_v, den_v, den1_v, h_v, out_v,
                     stripe_v, out_sh):
    cid, sid, wid = _ids()
    pltpu.sync_copy(src_h.at[wid], src_v)
    pltpu.sync_copy(dst_h.at[wid], dst_v)
    pltpu.sync_copy(ex_h.at[wid], ex_v)
    pltpu.sync_copy(den_h.at[0], den_v)
    pltpu.sync_copy(den_h.at[1], den1_v)
    pltpu.sync_copy(h_h, h_v)

    def _init(i, c):
        sl = pl.ds(i * 16, 16)
        den_v[sl] = den_v[sl] + den1_v[sl]
        out_v[sl] = _zero16()
        return c
    lax.fori_loop(0, NPAD // 16, _init, 0)

    def _step(i, c):
        sl = pl.ds(i * 16, 16)
        s16 = src_v[sl]
        d16 = dst_v[sl]
        g = plsc.load_gather(den_v, [d16])
        al = ex_v[sl] / (g + EPS)
        hh = plsc.load_gather(h_v, [s16])
        plsc.addupdate_scatter(out_v, [d16], al * hh)
        return c
    lax.fori_loop(0, CE // 16, _step, 0)

    _stripe_merge(out_v, out_sh, sid, stripe_v)
    pltpu.sync_copy(out_v.at[pl.ds(0, STR)],
                    out_h.at[cid, pl.ds(sid * STR, STR)])


_phase_b_sc = pl.kernel(
    _phase_b_sc_body,
    out_type=jax.ShapeDtypeStruct((NC, NPAD), jnp.float32),
    mesh=_mesh(),
    compiler_params=_sc_params,
    scratch_types=[
        pltpu.VMEM((CE,), jnp.int32),
        pltpu.VMEM((CE,), jnp.int32),
        pltpu.VMEM((CE,), jnp.float32),
        pltpu.VMEM((NPAD,), jnp.float32),
        pltpu.VMEM((NPAD,), jnp.float32),
        pltpu.VMEM((NPAD,), jnp.float32),
        pltpu.VMEM((NPAD,), jnp.float32),
        pltpu.VMEM((NS, STR), jnp.float32),
        pltpu.VMEM_SHARED((NS, NPAD), jnp.float32),
    ],
)


def _tc_first_body(x_ref, w_ref, asd_ref, h_ref, av_ref):
    h = jnp.dot(x_ref[...], w_ref[...], preferred_element_type=jnp.float32)
    h_ref[...] = h
    av_ref[...] = jnp.dot(h, asd_ref[...], preferred_element_type=jnp.float32)


def _tc_next_body(p_ref, b_ref, w_ref, asd_ref, h_ref, av_ref):
    x = jnp.maximum(p_ref[0] + p_ref[1] + b_ref[...], 0.0)
    h = jnp.dot(x, w_ref[...], preferred_element_type=jnp.float32)
    h_ref[...] = h
    av_ref[...] = jnp.dot(h, asd_ref[...], preferred_element_type=jnp.float32)


def _tc_first(x, w, asd, hp):
    return pl.pallas_call(
        _tc_first_body,
        out_shape=(jax.ShapeDtypeStruct((NPAD, hp), jnp.float32),
                   jax.ShapeDtypeStruct((NPAD, 2), jnp.float32)),
    )(x, w, asd)


def _tc_next(p, b, w, asd, hp):
    return pl.pallas_call(
        _tc_next_body,
        out_shape=(jax.ShapeDtypeStruct((NPAD, hp), jnp.float32),
                   jax.ShapeDtypeStruct((NPAD, 2), jnp.float32)),
    )(p, b, w, asd)


def kernel(x, edge_index, edge_attr, W1, a_s1, a_d1, b1,
           W2, a_s2, a_d2, b2, W3, a_s3, a_d3, b3):
    n = x.shape[0]
    e = edge_index.shape[1]
    loop = jnp.arange(n, dtype=edge_index.dtype)
    fill = jnp.full((EPAD - e - n,), NPAD - 1, dtype=edge_index.dtype)
    src = jnp.concatenate([edge_index[0], loop, fill])
    dst2 = jnp.concatenate([edge_index[1], loop, fill]).reshape(NW, CE)
    src2 = src.reshape(NW, CE)
    src3 = src.reshape(NW, NCH, GC)

    x_pad = jnp.pad(x, ((0, NPAD - n), (0, 0)))
    w2 = jnp.pad(W2, ((0, 0), (0, 4)))
    b2p = jnp.pad(b2, (0, 4))
    w3 = jnp.pad(W3, ((0, 4), (0, 0)))

    h1, av1 = _tc_first(x_pad, W1, jnp.stack([a_s1, a_d1], axis=1), 48)
    ex1, den1 = _phase_a(src2, dst2, av1[:, 0], av1[:, 1])
    p1 = _phase_b_48(src3, dst2, ex1, den1, h1)

    asd2 = jnp.stack([jnp.pad(a_s2, (0, 4)), jnp.pad(a_d2, (0, 4))], axis=1)
    h2, av2 = _tc_next(p1, b1, w2, asd2, 64)
    ex2, den2 = _phase_a(src2, dst2, av2[:, 0], av2[:, 1])
    p2 = _phase_b_64(src3, dst2, ex2, den2, jnp.stack([h2, h2]))

    h3, av3 = _tc_next(p2, b2p, w3, jnp.stack([a_s3, a_d3], axis=1), 1)
    ex3, den3 = _phase_a(src2, dst2, av3[:, 0], av3[:, 1])
    p3 = _phase_b_sc(src2, dst2, ex3, den3, h3.reshape(NPAD))

    return (p3[0] + p3[1])[:n] + b3[0]


# speedup vs baseline: 1.0925x; 1.0925x over previous
import functools

import jax
import jax.numpy as jnp
from jax import lax
from jax.experimental import pallas as pl
from jax.experimental.pallas import tpu as pltpu
from jax.experimental.pallas import tpu_sc as plsc

NC, NS = 2, 16
NW = NC * NS
GC = 128
NCH = 82
CE = NCH * GC
EPAD = NW * CE
NPAD = 10240
STR = NPAD // NS
EPS = 1e-16

_mesh = functools.partial(plsc.VectorSubcoreMesh,
                          core_axis_name="c", subcore_axis_name="s")
_sc_params = pltpu.CompilerParams(needs_layout_passes=False,
                                  use_tc_tiling_on_sc=False)


def _ids():
    cid = lax.axis_index("c")
    sid = lax.axis_index("s")
    return cid, sid, sid * NC + cid


def _zero16():
    return jnp.zeros((16,), jnp.float32)


def _stripe_merge(part_v, sh, sid, stripe_v):
    pltpu.sync_copy(part_v, sh.at[sid])
    plsc.subcore_barrier()
    pltpu.sync_copy(sh.at[:, pl.ds(sid * STR, STR)], stripe_v)

    def _sum(i, c):
        sl = pl.ds(i * 16, 16)
        acc = stripe_v[0, sl]
        for t in range(1, NS):
            acc = acc + stripe_v[t, sl]
        part_v[sl] = acc
        return c
    lax.fori_loop(0, STR // 16, _sum, 0)


def _phase_a_body(src_h, dst_h, as_h, ad_h, ex_h, den_h,
                  src_v, dst_v, as_v, ad_v, den_v, ex_v, stripe_v, den_sh):
    cid, sid, wid = _ids()
    pltpu.sync_copy(src_h.at[wid], src_v)
    pltpu.sync_copy(dst_h.at[wid], dst_v)
    pltpu.sync_copy(as_h, as_v)
    pltpu.sync_copy(ad_h, ad_v)

    def _z(i, c):
        den_v[pl.ds(i * 16, 16)] = _zero16()
        return c
    lax.fori_loop(0, NPAD // 16, _z, 0)

    def _step(i, c):
        sl = pl.ds(i * 16, 16)
        s16 = src_v[sl]
        d16 = dst_v[sl]
        a = plsc.load_gather(as_v, [s16])
        b = plsc.load_gather(ad_v, [d16])
        t = a + b
        ex = jnp.exp(jnp.where(t >= 0, t, 0.2 * t))
        ex_v[sl] = ex
        plsc.addupdate_scatter(den_v, [d16], ex)
        return c
    lax.fori_loop(0, CE // 16, _step, 0)

    pltpu.sync_copy(ex_v, ex_h.at[wid])
    _stripe_merge(den_v, den_sh, sid, stripe_v)
    pltpu.sync_copy(den_v.at[pl.ds(0, STR)],
                    den_h.at[cid, pl.ds(sid * STR, STR)])


_phase_a = pl.kernel(
    _phase_a_body,
    out_type=(jax.ShapeDtypeStruct((NW, CE), jnp.float32),
              jax.ShapeDtypeStruct((NC, NPAD), jnp.float32)),
    mesh=_mesh(),
    compiler_params=_sc_params,
    scratch_types=[
        pltpu.VMEM((CE,), jnp.int32),
        pltpu.VMEM((CE,), jnp.int32),
        pltpu.VMEM((NPAD,), jnp.float32),
        pltpu.VMEM((NPAD,), jnp.float32),
        pltpu.VMEM((NPAD,), jnp.float32),
        pltpu.VMEM((CE,), jnp.float32),
        pltpu.VMEM((NS, STR), jnp.float32),
        pltpu.VMEM_SHARED((NS, NPAD), jnp.float32),
    ],
)


def _phase_b_vec_body(hp, stage_h, src3_h, dst_h, ex_h, den_h, h_h, out_h,
                      src3_v, dst_v, ex_v, den_v, den1_v, alpha_v, didx_v,
                      rows0_v, rows1_v, sem0, sem1, *shared):
    if stage_h:
        h_sh, out_sh = shared
    else:
        (out_sh,) = shared
    cid, sid, wid = _ids()
    pltpu.sync_copy(src3_h.at[wid], src3_v)
    pltpu.sync_copy(dst_h.at[wid], dst_v)
    pltpu.sync_copy(ex_h.at[wid], ex_v)
    pltpu.sync_copy(den_h.at[0], den_v)
    pltpu.sync_copy(den_h.at[1], den1_v)

    def _dsum(i, c):
        sl = pl.ds(i * 16, 16)
        den_v[sl] = den_v[sl] + den1_v[sl]
        return c
    lax.fori_loop(0, NPAD // 16, _dsum, 0)

    for e2 in range(GC):
        for c in range(hp // 16):
            rows0_v[e2, pl.ds(c * 16, 16)] = _zero16()
    for i in range(NPAD // GC // NS):
        pltpu.sync_copy(rows0_v, out_sh.at[pl.ds((sid * 5 + i) * GC, GC), :])
    if stage_h:
        @pl.when(sid == 0)
        def _():
            pltpu.sync_copy(h_h, h_sh)
        h_src = h_sh
    else:
        h_src = h_h
    plsc.subcore_barrier()

    rows = (rows0_v, rows1_v)
    sems = (sem0, sem1)
    pltpu.async_copy(h_src.at[src3_v.at[0]], rows0_v, sem0)

    def _pair(g, carry):
        for b in range(2):
            ch = g * 2 + b
            buf = rows[b]

            @pl.when(ch + 1 < NCH)
            def _():
                pltpu.async_copy(h_src.at[src3_v.at[ch + 1]],
                                 rows[1 - b], sems[1 - b])
            pltpu.make_async_copy(h_src.at[src3_v.at[ch]], buf, sems[b]).wait()

            for k in range(GC // 16):
                sl = pl.ds(k * 16, 16)
                esl = pl.ds(ch * GC + k * 16, 16)
                d16 = dst_v[esl]
                g16 = plsc.load_gather(den_v, [d16])
                alpha_v[sl] = ex_v[esl] / (g16 + EPS)
                didx_v[sl] = d16
            for k in range(GC // 16):
                al16 = alpha_v[pl.ds(k * 16, 16)]
                for l in range(16):
                    e2 = k * 16 + l
                    a16 = jnp.full((16,), al16[l], jnp.float32)
                    for c in range(hp // 16):
                        sl2 = pl.ds(c * 16, 16)
                        buf[e2, sl2] = buf[e2, sl2] * a16
            pltpu.sync_copy(buf, out_sh.at[didx_v], add=True)
        return carry
    lax.fori_loop(0, NCH // 2, _pair, 0)

    plsc.subcore_barrier()

    @pl.when(sid == 0)
    def _():
        pltpu.sync_copy(out_sh, out_h.at[cid])


def _make_phase_b_vec(hp, stage_h):
    shared = [pltpu.VMEM_SHARED((NPAD, hp), jnp.float32)]
    if stage_h:
        shared = [pltpu.VMEM_SHARED((NPAD, hp), jnp.float32)] + shared
    return pl.kernel(
        functools.partial(_phase_b_vec_body, hp, stage_h),
        out_type=jax.ShapeDtypeStruct((NC, NPAD, hp), jnp.float32),
        mesh=_mesh(),
        compiler_params=_sc_params,
        scratch_types=[
            pltpu.VMEM((NCH, GC), jnp.int32),
            pltpu.VMEM((CE,), jnp.int32),
            pltpu.VMEM((CE,), jnp.float32),
            pltpu.VMEM((NPAD,), jnp.float32),
            pltpu.VMEM((NPAD,), jnp.float32),
            pltpu.VMEM((GC,), jnp.float32),
            pltpu.VMEM((GC,), jnp.int32),
            pltpu.VMEM((GC, hp), jnp.float32),
            pltpu.VMEM((GC, hp), jnp.float32),
            pltpu.SemaphoreType.DMA,
            pltpu.SemaphoreType.DMA,
        ] + shared,
    )


_phase_b_48 = _make_phase_b_vec(48, True)
_phase_b_64 = _make_phase_b_vec(64, False)


def _phase_b_sc_body(src_h, dst_h, ex_h, den_h, h_h, out_h,
                     src_v, dst_v, ex_v, den_v, den1_v, h_v, out_v,
                     stripe_v, out_sh):
    cid, sid, wid = _ids()
    pltpu.sync_copy(src_h.at[wid], src_v)
    pltpu.sync_copy(dst_h.at[wid], dst_v)
    pltpu.sync_copy(ex_h.at[wid], ex_v)
    pltpu.sync_copy(den_h.at[0], den_v)
    pltpu.sync_copy(den_h.at[1], den1_v)
    pltpu.sync_copy(h_h, h_v)

    def _init(i, c):
        sl = pl.ds(i * 16, 16)
        den_v[sl] = den_v[sl] + den1_v[sl]
        out_v[sl] = _zero16()
        return c
    lax.fori_loop(0, NPAD // 16, _init, 0)

    def _step(i, c):
        sl = pl.ds(i * 16, 16)
        s16 = src_v[sl]
        d16 = dst_v[sl]
        g = plsc.load_gather(den_v, [d16])
        al = ex_v[sl] / (g + EPS)
        hh = plsc.load_gather(h_v, [s16])
        plsc.addupdate_scatter(out_v, [d16], al * hh)
        return c
    lax.fori_loop(0, CE // 16, _step, 0)

    _stripe_merge(out_v, out_sh, sid, stripe_v)
    pltpu.sync_copy(out_v.at[pl.ds(0, STR)],
                    out_h.at[cid, pl.ds(sid * STR, STR)])


_phase_b_sc = pl.kernel(
    _phase_b_sc_body,
    out_type=jax.ShapeDtypeStruct((NC, NPAD), jnp.float32),
    mesh=_mesh(),
    compiler_params=_sc_params,
    scratch_types=[
        pltpu.VMEM((CE,), jnp.int32),
        pltpu.VMEM((CE,), jnp.int32),
        pltpu.VMEM((CE,), jnp.float32),
        pltpu.VMEM((NPAD,), jnp.float32),
        pltpu.VMEM((NPAD,), jnp.float32),
        pltpu.VMEM((NPAD,), jnp.float32),
        pltpu.VMEM((NPAD,), jnp.float32),
        pltpu.VMEM((NS, STR), jnp.float32),
        pltpu.VMEM_SHARED((NS, NPAD), jnp.float32),
    ],
)


def _tc_first_body(x_ref, w_ref, asd_ref, h_ref, av_ref):
    h = jnp.dot(x_ref[...], w_ref[...], preferred_element_type=jnp.float32)
    h_ref[...] = h
    av_ref[...] = jnp.dot(h, asd_ref[...], preferred_element_type=jnp.float32)


def _tc_next_body(p_ref, b_ref, w_ref, asd_ref, h_ref, av_ref):
    x = jnp.maximum(p_ref[0] + p_ref[1] + b_ref[...], 0.0)
    h = jnp.dot(x, w_ref[...], preferred_element_type=jnp.float32)
    h_ref[...] = h
    av_ref[...] = jnp.dot(h, asd_ref[...], preferred_element_type=jnp.float32)


def _tc_first(x, w, asd, hp):
    return pl.pallas_call(
        _tc_first_body,
        out_shape=(jax.ShapeDtypeStruct((NPAD, hp), jnp.float32),
                   jax.ShapeDtypeStruct((NPAD, 2), jnp.float32)),
    )(x, w, asd)


def _tc_next(p, b, w, asd, hp):
    return pl.pallas_call(
        _tc_next_body,
        out_shape=(jax.ShapeDtypeStruct((NPAD, hp), jnp.float32),
                   jax.ShapeDtypeStruct((NPAD, 2), jnp.float32)),
    )(p, b, w, asd)


def kernel(x, edge_index, edge_attr, W1, a_s1, a_d1, b1,
           W2, a_s2, a_d2, b2, W3, a_s3, a_d3, b3):
    n = x.shape[0]
    e = edge_index.shape[1]
    loop = jnp.arange(n, dtype=edge_index.dtype)
    fill = jnp.full((EPAD - e - n,), NPAD - 1, dtype=edge_index.dtype)
    src = jnp.concatenate([edge_index[0], loop, fill])
    dst2 = jnp.concatenate([edge_index[1], loop, fill]).reshape(NW, CE)
    src2 = src.reshape(NW, CE)
    src3 = src.reshape(NW, NCH, GC)

    x_pad = jnp.pad(x, ((0, NPAD - n), (0, 0)))
    w2 = jnp.pad(W2, ((0, 0), (0, 4)))
    b2p = jnp.pad(b2, (0, 4))
    w3 = jnp.pad(W3, ((0, 4), (0, 0)))

    h1, av1 = _tc_first(x_pad, W1, jnp.stack([a_s1, a_d1], axis=1), 48)
    ex1, den1 = _phase_a(src2, dst2, av1[:, 0], av1[:, 1])
    p1 = _phase_b_48(src3, dst2, ex1, den1, h1)

    asd2 = jnp.stack([jnp.pad(a_s2, (0, 4)), jnp.pad(a_d2, (0, 4))], axis=1)
    h2, av2 = _tc_next(p1, b1, w2, asd2, 64)
    ex2, den2 = _phase_a(src2, dst2, av2[:, 0], av2[:, 1])
    p2 = _phase_b_64(src3, dst2, ex2, den2, h2)

    h3, av3 = _tc_next(p2, b2p, w3, jnp.stack([a_s3, a_d3], axis=1), 1)
    ex3, den3 = _phase_a(src2, dst2, av3[:, 0], av3[:, 1])
    p3 = _phase_b_sc(src2, dst2, ex3, den3, h3.reshape(NPAD))

    return (p3[0] + p3[1])[:n] + b3[0]
